# baseline (speedup 1.0000x reference)
import functools

import jax
import jax.numpy as jnp
from jax.experimental import pallas as pl
from jax.experimental.pallas import tpu as pltpu

NUM_CODES = 1024
NUM_CHANNELS = 64
BT = 2048
NBUF = 6


def _vq_logits_kernel(n_steps, kt_ref, et_ref, out_ref, scratch, sems):
    i = pl.program_id(0)
    slot = jax.lax.rem(i, NBUF)

    @pl.when(i >= NBUF)
    def _wait_prev():
        pltpu.make_async_copy(
            scratch.at[slot],
            out_ref.at[pl.ds((i - NBUF) * BT, BT), :],
            sems.at[slot],
        ).wait()

    kt = kt_ref[...]
    et = et_ref[...]
    k_sq = jnp.sum(kt * kt, axis=0, keepdims=True)
    e_sq = jnp.sum(et * et, axis=0, keepdims=True)
    a_t = jnp.concatenate(
        [kt + kt, -k_sq, jnp.ones_like(k_sq)], axis=0
    )
    b_t = jnp.concatenate(
        [et, jnp.ones_like(e_sq), -e_sq], axis=0
    )
    scratch[slot] = jax.lax.dot_general(
        a_t, b_t, (((0,), (0,)), ((), ())),
        preferred_element_type=jnp.float32,
    )

    pltpu.make_async_copy(
        scratch.at[slot],
        out_ref.at[pl.ds(i * BT, BT), :],
        sems.at[slot],
    ).start()

    @pl.when(i == n_steps - 1)
    def _drain():
        for s in range(NBUF):
            pltpu.make_async_copy(
                scratch.at[s],
                out_ref.at[pl.ds(0, BT), :],
                sems.at[s],
            ).wait()


@jax.jit
def kernel(keys, embeddings):
    n_tokens = keys.shape[0]
    n_steps = n_tokens // BT
    kt = keys.T
    et = embeddings.T
    return pl.pallas_call(
        functools.partial(_vq_logits_kernel, n_steps),
        grid=(n_steps,),
        in_specs=[
            pl.BlockSpec((NUM_CHANNELS, BT), lambda i: (0, i)),
            pl.BlockSpec((NUM_CHANNELS, NUM_CODES), lambda i: (0, 0)),
        ],
        out_specs=pl.BlockSpec(memory_space=pl.ANY),
        out_shape=jax.ShapeDtypeStruct((n_tokens, NUM_CODES), jnp.float32),
        scratch_shapes=[
            pltpu.VMEM((NBUF, BT, NUM_CODES), jnp.float32),
            pltpu.SemaphoreType.DMA((NBUF,)),
        ],
        compiler_params=pltpu.CompilerParams(
            dimension_semantics=("arbitrary",),
        ),
    )(kt, et)


# speedup vs baseline: 1.0020x; 1.0020x over previous
import functools

import jax
import jax.numpy as jnp
from jax.experimental import pallas as pl
from jax.experimental.pallas import tpu as pltpu

NUM_CODES = 1024
NUM_CHANNELS = 64
BT = 2048
NBUF = 4


def _vq_logits_kernel(n_steps, kt_ref, et_ref, out_ref, scratch, sems):
    i = pl.program_id(0)
    slot = jax.lax.rem(i, NBUF)

    @pl.when(i >= NBUF)
    def _wait_prev():
        pltpu.make_async_copy(
            scratch.at[slot],
            out_ref.at[pl.ds((i - NBUF) * BT, BT), :],
            sems.at[slot],
        ).wait()

    kt = kt_ref[...]
    et = et_ref[...]
    k_sq = jnp.sum(kt * kt, axis=0, keepdims=True)
    e_sq = jnp.sum(et * et, axis=0, keepdims=True)
    a_t = jnp.concatenate(
        [kt + kt, -k_sq, jnp.ones_like(k_sq)], axis=0
    )
    b_t = jnp.concatenate(
        [et, jnp.ones_like(e_sq), -e_sq], axis=0
    )
    scratch[slot] = jax.lax.dot_general(
        a_t, b_t, (((0,), (0,)), ((), ())),
        preferred_element_type=jnp.float32,
    )

    pltpu.make_async_copy(
        scratch.at[slot],
        out_ref.at[pl.ds(i * BT, BT), :],
        sems.at[slot],
    ).start()

    @pl.when(i == n_steps - 1)
    def _drain():
        for s in range(NBUF):
            pltpu.make_async_copy(
                scratch.at[s],
                out_ref.at[pl.ds(0, BT), :],
                sems.at[s],
            ).wait()


@jax.jit
def kernel(keys, embeddings):
    n_tokens = keys.shape[0]
    n_steps = n_tokens // BT
    kt = keys.T
    et = embeddings.T
    return pl.pallas_call(
        functools.partial(_vq_logits_kernel, n_steps),
        grid=(n_steps,),
        in_specs=[
            pl.BlockSpec((NUM_CHANNELS, BT), lambda i: (0, i)),
            pl.BlockSpec((NUM_CHANNELS, NUM_CODES), lambda i: (0, 0)),
        ],
        out_specs=pl.BlockSpec(memory_space=pl.ANY),
        out_shape=jax.ShapeDtypeStruct((n_tokens, NUM_CODES), jnp.float32),
        scratch_shapes=[
            pltpu.VMEM((NBUF, BT, NUM_CODES), jnp.float32),
            pltpu.SemaphoreType.DMA((NBUF,)),
        ],
        compiler_params=pltpu.CompilerParams(
            dimension_semantics=("arbitrary",),
        ),
    )(kt, et)


# speedup vs baseline: 1.0103x; 1.0084x over previous
import functools

import jax
import jax.numpy as jnp
from jax.experimental import pallas as pl
from jax.experimental.pallas import tpu as pltpu

NUM_CODES = 1024
NUM_CHANNELS = 64
BT = 2048
BTH = 1024
NBUF = 4


def _vq_logits_kernel(n_steps, kt_ref, et_ref, out_ref, scratch, sems):
    i = pl.program_id(0)

    kt = kt_ref[...]
    et = et_ref[...]
    k_sq = jnp.sum(kt * kt, axis=0, keepdims=True)
    e_sq = jnp.sum(et * et, axis=0, keepdims=True)
    a_t = jnp.concatenate(
        [kt + kt, -k_sq, jnp.ones_like(k_sq)], axis=0
    )
    b_t = jnp.concatenate(
        [et, jnp.ones_like(e_sq), -e_sq], axis=0
    )

    for h in range(2):
        c = 2 * i + h
        slot = jax.lax.rem(c, NBUF)

        @pl.when(c >= NBUF)
        def _wait_prev(slot=slot, c=c):
            pltpu.make_async_copy(
                scratch.at[slot],
                out_ref.at[pl.ds((c - NBUF) * BTH, BTH), :],
                sems.at[slot],
            ).wait()

        scratch[slot] = jax.lax.dot_general(
            a_t[:, h * BTH:(h + 1) * BTH], b_t,
            (((0,), (0,)), ((), ())),
            preferred_element_type=jnp.float32,
        )

        pltpu.make_async_copy(
            scratch.at[slot],
            out_ref.at[pl.ds(c * BTH, BTH), :],
            sems.at[slot],
        ).start()

    @pl.when(i == n_steps - 1)
    def _drain():
        for s in range(NBUF):
            pltpu.make_async_copy(
                scratch.at[s],
                out_ref.at[pl.ds(0, BTH), :],
                sems.at[s],
            ).wait()


@jax.jit
def kernel(keys, embeddings):
    n_tokens = keys.shape[0]
    n_steps = n_tokens // BT
    kt = keys.T
    et = embeddings.T
    return pl.pallas_call(
        functools.partial(_vq_logits_kernel, n_steps),
        grid=(n_steps,),
        in_specs=[
            pl.BlockSpec((NUM_CHANNELS, BT), lambda i: (0, i)),
            pl.BlockSpec((NUM_CHANNELS, NUM_CODES), lambda i: (0, 0)),
        ],
        out_specs=pl.BlockSpec(memory_space=pl.ANY),
        out_shape=jax.ShapeDtypeStruct((n_tokens, NUM_CODES), jnp.float32),
        scratch_shapes=[
            pltpu.VMEM((NBUF, BTH, NUM_CODES), jnp.float32),
            pltpu.SemaphoreType.DMA((NBUF,)),
        ],
        compiler_params=pltpu.CompilerParams(
            dimension_semantics=("arbitrary",),
        ),
    )(kt, et)
